# baseline (speedup 1.0000x reference)
import jax
import jax.numpy as jnp
from jax.experimental import pallas as pl
from jax.experimental.pallas import tpu as pltpu

INPUT_DIM = 832
HIDDEN1 = 256
HIDDEN2 = 32
BATCH = 16384
BN = 2048
NP = BATCH // BN
NBUF = 3
HALF = BATCH // 2


def _mlp_pipe(xt_ref, w1t_ref, b1_ref, w2_ref, b2_ref, w3_ref, b3_ref,
              out_ref, xbuf_ref, hbuf_ref, sem_ref):
    def start(p):
        pltpu.make_async_copy(
            xt_ref.at[:, pl.ds(p * BN, BN)], xbuf_ref.at[p % NBUF],
            sem_ref.at[p % NBUF]).start()

    def wait(p):
        pltpu.make_async_copy(
            xt_ref.at[:, pl.ds(p * BN, BN)], xbuf_ref.at[p % NBUF],
            sem_ref.at[p % NBUF]).wait()

    def tail_half(h):
        off = h * HALF
        hh = hbuf_ref[:, pl.ds(off, HALF)]
        g = jax.lax.dot_general(
            w2_ref[...].astype(jnp.bfloat16), hh, (((1,), (0,)), ((), ())),
            preferred_element_type=jnp.float32)
        g = jnp.maximum(g + b2_ref[...], 0.0)
        out_ref[0:1, pl.ds(off, HALF)] = (
            jnp.sum(g * w3_ref[...], axis=0, keepdims=True) + b3_ref[0, 0])

    w1b = w1t_ref[...].astype(jnp.bfloat16)

    for p in range(min(NBUF, NP)):
        start(p)
    for p in range(NP):
        wait(p)
        xb = xbuf_ref[p % NBUF].astype(jnp.bfloat16)
        hp = jax.lax.dot_general(
            w1b, xb, (((0,), (0,)), ((), ())),
            preferred_element_type=jnp.float32)
        hp = jnp.maximum(hp + b1_ref[...], 0.0)
        hbuf_ref[:, pl.ds(p * BN, BN)] = hp.astype(jnp.bfloat16)
        if p + NBUF < NP:
            start(p + NBUF)
        if p == NP // 2 + 1:
            tail_half(0)
    tail_half(1)


def kernel(x, w1, b1, w2, b2, w3, b3):
    xt = x.T
    w1t = w1.T
    b1c = b1.reshape(HIDDEN1, 1)
    b2c = b2.reshape(HIDDEN2, 1)
    w3c = w3.reshape(HIDDEN2, 1)
    b3r = b3.reshape(1, 1)

    vfull = lambda: pl.BlockSpec(memory_space=pl.ANY)
    outt = pl.pallas_call(
        _mlp_pipe,
        in_specs=[
            pl.BlockSpec(memory_space=pl.ANY),
            pl.BlockSpec((INPUT_DIM, HIDDEN1), lambda: (0, 0)),
            pl.BlockSpec((HIDDEN1, 1), lambda: (0, 0)),
            pl.BlockSpec((HIDDEN2, HIDDEN1), lambda: (0, 0)),
            pl.BlockSpec((HIDDEN2, 1), lambda: (0, 0)),
            pl.BlockSpec((HIDDEN2, 1), lambda: (0, 0)),
            pl.BlockSpec((1, 1), lambda: (0, 0)),
        ],
        out_specs=pl.BlockSpec((1, BATCH), lambda: (0, 0)),
        out_shape=jax.ShapeDtypeStruct((1, BATCH), jnp.float32),
        scratch_shapes=[
            pltpu.VMEM((NBUF, INPUT_DIM, BN), jnp.float32),
            pltpu.VMEM((HIDDEN1, BATCH), jnp.bfloat16),
            pltpu.SemaphoreType.DMA((NBUF,)),
        ],
    )(xt, w1t, b1c, w2, b2c, w3c, b3r)
    return outt.reshape(BATCH, 1)
